# baseline (speedup 1.0000x reference)
import functools

import jax
import jax.numpy as jnp
from jax import lax
from jax.experimental import pallas as pl
from jax.experimental.pallas import tpu as pltpu
from jax.experimental.pallas import tpu_sc as plsc

N = 50000
E = 800000
B = 100
L = 500
HID = 96
IN_DIM = 300
NCLS = 20

NC = 2
NS = 16

NCHUNK = 6
CH = 8352
ACC_ROWS = CH + 32
NOUT = NCHUNK * CH
ZR = 131
E_PAD = 819200
EPT = E_PAD // NS
KE = 512
KSUB = KE // 128
NB = EPT // KE

IDS_PAD = 53248
IPT = IDS_PAD // (NC * NS)
NBI = IPT // 128
EMB_D = 304

RB = 1000


def _sc_mesh():
    return plsc.VectorSubcoreMesh(core_axis_name="c", subcore_axis_name="s")


_SC_PARAMS = pltpu.CompilerParams(use_tc_tiling_on_sc=False,
                                  internal_scratch_in_bytes=0)


def _embed_gather(embed, ids2):
    @functools.partial(
        pl.kernel,
        out_type=jax.ShapeDtypeStruct((IDS_PAD, EMB_D), jnp.float32),
        mesh=_sc_mesh(),
        scratch_types=[
            pltpu.VMEM((NBI, 128), jnp.int32),
            pltpu.VMEM((128, EMB_D), jnp.float32),
            pltpu.SemaphoreType.DMA,
        ],
        compiler_params=_SC_PARAMS,
    )
    def k(table, ids, out, idx_v, rows_v, sem):
        wid = lax.axis_index("s") * NC + lax.axis_index("c")
        pltpu.sync_copy(ids.at[pl.ds(wid * NBI, NBI)], idx_v)

        def body(b, carry):
            pltpu.async_copy(table.at[idx_v.at[b]], rows_v, sem).wait()
            pltpu.sync_copy(rows_v, out.at[pl.ds(wid * IPT + b * 128, 128)])
            return carry

        lax.fori_loop(0, NBI, body, 0)

    return k(embed, ids2)


def _edge_aggregate(x, src, dst, w):
    @functools.partial(
        pl.kernel,
        out_type=jax.ShapeDtypeStruct((NOUT, HID), jnp.float32),
        mesh=_sc_mesh(),
        scratch_types=[
            pltpu.VMEM((KSUB, 128), jnp.int32),
            pltpu.VMEM((KE,), jnp.int32),
            pltpu.VMEM((KSUB, 128), jnp.int32),
            pltpu.VMEM((KE, HID), jnp.float32),
            pltpu.VMEM((ZR, HID), jnp.float32),
            pltpu.VMEM_SHARED((ACC_ROWS, HID), jnp.float32),
            pltpu.VMEM((KE,), jnp.float32),
            pltpu.SemaphoreType.DMA,
        ],
        compiler_params=_SC_PARAMS,
    )
    def k(x_hbm, src_hbm, dst_hbm, w_hbm, out_hbm,
          src_v, dst_v, idx_v, rows_v, zero_v, acc, w_v, sem):
        cid = lax.axis_index("c")
        sid = lax.axis_index("s")

        def zrow(r, carry):
            for j in range(HID // 16):
                zero_v[r, pl.ds(j * 16, 16)] = jnp.zeros((16,), jnp.float32)
            return carry

        lax.fori_loop(0, ZR, zrow, 0)

        for cc in range(NCHUNK // NC):
            rbase = (cid * (NCHUNK // NC) + cc) * CH
            for z in range(4):
                pltpu.sync_copy(zero_v, acc.at[pl.ds(sid * (4 * ZR) + z * ZR, ZR)])
            plsc.subcore_barrier()

            def blk(b, carry):
                off = sid * EPT + b * KE
                pltpu.sync_copy(src_hbm.at[pl.ds(off // 128, KSUB)], src_v)
                pltpu.sync_copy(dst_hbm.at[pl.ds(off, KE)], dst_v)
                pltpu.sync_copy(w_hbm.at[pl.ds(off, KE)], w_v)
                copies = [
                    pltpu.async_copy(x_hbm.at[src_v.at[j]],
                                     rows_v.at[pl.ds(j * 128, 128)], sem)
                    for j in range(KSUB)
                ]
                for c in copies:
                    c.wait()

                def grp(g, c2):
                    e0 = g * 16
                    d = dst_v[pl.ds(e0, 16)]
                    dl = d - rbase
                    ok = (dl >= 0) & (dl < CH)
                    idx_v[g // 8, pl.ds((g % 8) * 16, 16)] = jnp.where(ok, dl, CH)
                    wv = w_v[pl.ds(e0, 16)]
                    for e in range(16):
                        wb = wv.at[jnp.full((16,), e, jnp.int32)].get(
                            mode="promise_in_bounds")
                        for j in range(HID // 16):
                            rows_v[e0 + e, pl.ds(j * 16, 16)] = (
                                rows_v[e0 + e, pl.ds(j * 16, 16)] * wb)
                    return c2

                lax.fori_loop(0, KE // 16, grp, 0)
                for j in range(KSUB):
                    pltpu.sync_copy(rows_v.at[pl.ds(j * 128, 128)],
                                    acc.at[idx_v.at[j]], add=True)
                return carry

            lax.fori_loop(0, NB, blk, 0)
            plsc.subcore_barrier()
            pltpu.sync_copy(acc.at[pl.ds(sid * (CH // NS), CH // NS)],
                            out_hbm.at[pl.ds(rbase + sid * (CH // NS), CH // NS)])
            plsc.subcore_barrier()

    return k(x, src, dst, w)


def _encode(x300, W_enc, b_enc):
    def body(x_ref, w_ref, b_ref, o_ref):
        o_ref[...] = jnp.tanh(
            jnp.dot(x_ref[...], w_ref[...], preferred_element_type=jnp.float32)
            + b_ref[...]
        )

    return pl.pallas_call(
        body,
        out_shape=jax.ShapeDtypeStruct((N, HID), jnp.float32),
        grid=(N // RB,),
        in_specs=[
            pl.BlockSpec((RB, EMB_D), lambda i: (i, 0)),
            pl.BlockSpec((EMB_D, HID), lambda i: (0, 0)),
            pl.BlockSpec((HID,), lambda i: (0,)),
        ],
        out_specs=pl.BlockSpec((RB, HID), lambda i: (i, 0)),
    )(x300, W_enc, b_enc)


def _gru(a, x, Wz0, Wz1, Wr0, Wr1, Wh0, Wh1, bz, br, bh):
    def body(a_ref, x_ref, wz0, wz1, wr0, wr1, wh0, wh1, bz_r, br_r, bh_r, o_ref):
        ab = a_ref[...]
        xb = x_ref[...]
        dot = lambda p, q: jnp.dot(p, q, preferred_element_type=jnp.float32)
        zg = jax.nn.sigmoid(dot(ab, wz0[...]) + dot(xb, wz1[...]) + bz_r[...])
        rg = jax.nn.sigmoid(dot(ab, wr0[...]) + dot(xb, wr1[...]) + br_r[...])
        hg = jnp.tanh(dot(ab, wh0[...]) + dot(xb * rg, wh1[...]) + bh_r[...])
        o_ref[...] = hg * zg + xb * (1.0 - zg)

    wspec = pl.BlockSpec((HID, HID), lambda i: (0, 0))
    bspec = pl.BlockSpec((HID,), lambda i: (0,))
    return pl.pallas_call(
        body,
        out_shape=jax.ShapeDtypeStruct((N, HID), jnp.float32),
        grid=(N // RB,),
        in_specs=[
            pl.BlockSpec((RB, HID), lambda i: (i, 0)),
            pl.BlockSpec((RB, HID), lambda i: (i, 0)),
            wspec, wspec, wspec, wspec, wspec, wspec,
            bspec, bspec, bspec,
        ],
        out_specs=pl.BlockSpec((RB, HID), lambda i: (i, 0)),
    )(a, x, Wz0, Wz1, Wr0, Wr1, Wh0, Wh1, bz, br, bh)


def _readout_body(xb_ref, watt_ref, batt_ref, wemb_ref, bemb_ref, wmlp_ref, bmlp_ref, out_ref):
    xb = xb_ref[0]
    att = jax.nn.sigmoid(
        lax.dot_general(xb, watt_ref[...], (((1,), (0,)), ((), ()))) + batt_ref[...]
    )
    emb = jnp.tanh(
        lax.dot_general(xb, wemb_ref[...], (((1,), (0,)), ((), ()))) + bemb_ref[...]
    )
    xv = att * emb
    xmax = jnp.max(xv, axis=0)
    xmean = jnp.sum(xv, axis=0) / float(L)
    red = (xmax + xmean)[None, :]
    out_ref[0] = (
        lax.dot_general(red, wmlp_ref[...], (((1,), (0,)), ((), ()))) + bmlp_ref[...]
    )


def _readout(xb, W_att, b_att, W_emb, b_emb, W_mlp, b_mlp):
    return pl.pallas_call(
        _readout_body,
        out_shape=jax.ShapeDtypeStruct((B, 1, NCLS), jnp.float32),
        grid=(B,),
        in_specs=[
            pl.BlockSpec((1, L, HID), lambda b: (b, 0, 0)),
            pl.BlockSpec((HID, 1), lambda b: (0, 0)),
            pl.BlockSpec((1,), lambda b: (0,)),
            pl.BlockSpec((HID, HID), lambda b: (0, 0)),
            pl.BlockSpec((HID,), lambda b: (0,)),
            pl.BlockSpec((HID, NCLS), lambda b: (0, 0)),
            pl.BlockSpec((NCLS,), lambda b: (0,)),
        ],
        out_specs=pl.BlockSpec((1, 1, NCLS), lambda b: (b, 0, 0)),
    )(xb, W_att, b_att, W_emb, b_emb, W_mlp, b_mlp).reshape(B, NCLS)


def kernel(x_ids, edge_index, edge_attr, length, embed,
           W_enc, b_enc, Wz0, bz0, Wz1, bz1, Wr0, br0, Wr1, br1,
           Wh0, bh0, Wh1, bh1, W_att, b_att, W_emb, b_emb, W_mlp, b_mlp):
    ids2 = jnp.concatenate(
        [x_ids.astype(jnp.int32), jnp.zeros((IDS_PAD - N,), jnp.int32)]
    ).reshape(IDS_PAD // 128, 128)
    embed_p = jnp.pad(embed, ((0, 0), (0, EMB_D - IN_DIM)))
    W_enc_p = jnp.pad(W_enc, ((0, EMB_D - IN_DIM), (0, 0)))
    x300 = _embed_gather(embed_p, ids2)[:N]
    x = _encode(x300, W_enc_p, b_enc)
    epad = E_PAD - E
    src2 = jnp.concatenate(
        [edge_index[0].astype(jnp.int32), jnp.zeros((epad,), jnp.int32)]
    ).reshape(E_PAD // 128, 128)
    dst_p = jnp.concatenate(
        [edge_index[1].astype(jnp.int32), jnp.full((epad,), N, jnp.int32)])
    w_p = jnp.concatenate([edge_attr, jnp.zeros((epad,), jnp.float32)])
    bz = bz0 + bz1
    br = br0 + br1
    bh = bh0 + bh1
    for _ in range(2):
        a = _edge_aggregate(x, src2, dst_p, w_p)[:N]
        x = _gru(a, x, Wz0, Wz1, Wr0, Wr1, Wh0, Wh1, bz, br, bh)
    xb = x.reshape(B, L, HID)
    return _readout(xb, W_att, b_att, W_emb, b_emb, W_mlp, b_mlp)
